# Initial kernel scaffold; baseline (speedup 1.0000x reference)
import jax
import jax.numpy as jnp
from jax.experimental import pallas as pl


def kernel(x, edge_index, batch, W1_0, b1_0, W2_0, b2_0, W1_1, b1_1, W2_1, b2_1, W1_2, b1_2, W2_2, b2_2, W_out, b_out):
    raise NotImplementedError("write your pallas kernel here")


# speedup vs baseline: 6.4405x; 6.4405x over previous
import functools

import jax
import jax.numpy as jnp
from jax import lax
from jax.experimental import pallas as pl
from jax.experimental.pallas import tpu as pltpu
from jax.experimental.pallas import tpu_sc as plsc

N = 10000
E = 320000
D = 128
OUT = 128
NUM_GRAPHS = 64

NC = 2
NS = 16
NW = NC * NS
CHUNK = 80
ROWS_PER_W = E // CHUNK // NW
N_PAD = 10240
NODES_PER_S = N_PAD // NS


def _seg_sum_body(h_hbm, src_hbm, dst_hbm, out_hbm,
                  src_v, dst_v, rows_v, agg_s, gsem):
    c = lax.axis_index("c")
    s = lax.axis_index("s")
    wid = c * NS + s

    zvec = jnp.zeros((16,), jnp.float32)
    def _zrow(i, carry):
        for j in range(D // 16):
            rows_v[i, pl.ds(j * 16, 16)] = zvec
        return carry
    lax.fori_loop(0, CHUNK, _zrow, 0)
    for k in range(NODES_PER_S // CHUNK):
        pltpu.sync_copy(rows_v, agg_s.at[pl.ds(s * NODES_PER_S + k * CHUNK, CHUNK)])
    plsc.subcore_barrier()

    pltpu.sync_copy(src_hbm.at[wid], src_v)
    pltpu.sync_copy(dst_hbm.at[wid], dst_v)

    def _edge(j, carry):
        pltpu.async_copy(h_hbm.at[src_v.at[j]], rows_v, gsem).wait()
        pltpu.sync_copy(rows_v, agg_s.at[dst_v.at[j]], add=True)
        return carry
    lax.fori_loop(0, ROWS_PER_W, _edge, 0)
    plsc.subcore_barrier()

    pltpu.sync_copy(agg_s.at[pl.ds(s * NODES_PER_S, NODES_PER_S)],
                    out_hbm.at[c, pl.ds(s * NODES_PER_S, NODES_PER_S)])


_seg_sum = pl.kernel(
    _seg_sum_body,
    out_type=jax.ShapeDtypeStruct((NC, N_PAD, D), jnp.float32),
    mesh=plsc.VectorSubcoreMesh(core_axis_name="c", subcore_axis_name="s"),
    scratch_types=[
        pltpu.VMEM((ROWS_PER_W, CHUNK), jnp.int32),
        pltpu.VMEM((ROWS_PER_W, CHUNK), jnp.int32),
        pltpu.VMEM((CHUNK, D), jnp.float32),
        pltpu.VMEM_SHARED((N_PAD, D), jnp.float32),
        pltpu.SemaphoreType.DMA,
    ],
)


def _mlp_body(h_ref, p0_ref, p1_ref, w1_ref, b1_ref, w2_ref, b2_ref, o_ref,
              *, relu_out):
    m = h_ref[...] + p0_ref[...] + p1_ref[...]
    z = jnp.maximum(
        jnp.dot(m, w1_ref[...], preferred_element_type=jnp.float32)
        + b1_ref[...], 0.0)
    y = (jnp.dot(z, w2_ref[...], preferred_element_type=jnp.float32)
         + b2_ref[...])
    o_ref[...] = jnp.maximum(y, 0.0) if relu_out else y


BLK = 1000


def _mlp(h, p0, p1, w1, b1, w2, b2, relu_out):
    grid = (N // BLK,)
    row_spec = pl.BlockSpec((BLK, D), lambda i: (i, 0))
    full = lambda shape: pl.BlockSpec(shape, lambda i: (0, 0))
    return pl.pallas_call(
        functools.partial(_mlp_body, relu_out=relu_out),
        grid=grid,
        in_specs=[row_spec, row_spec, row_spec,
                  full((D, D)), full((1, D)), full((D, D)), full((1, D))],
        out_specs=row_spec,
        out_shape=jax.ShapeDtypeStruct((N, D), jnp.float32),
    )(h, p0, p1, w1, b1.reshape(1, D), w2, b2.reshape(1, D))


def _pool_body(h_ref, batch_ref, wout_ref, bout_ref, out_ref, emb_ref):
    oh = (batch_ref[...]
          == lax.broadcasted_iota(jnp.int32, (1, NUM_GRAPHS), 1)
          ).astype(jnp.float32)
    emb = lax.dot_general(oh, h_ref[...], (((0,), (0,)), ((), ())),
                          preferred_element_type=jnp.float32)
    emb_ref[...] = emb
    out_ref[...] = (jnp.dot(emb, wout_ref[...],
                            preferred_element_type=jnp.float32)
                    + bout_ref[...])


def _pool(h, batch2d, w_out, b_out):
    return pl.pallas_call(
        _pool_body,
        in_specs=[pl.BlockSpec((N, D), lambda: (0, 0)),
                  pl.BlockSpec((N, 1), lambda: (0, 0)),
                  pl.BlockSpec((D, OUT), lambda: (0, 0)),
                  pl.BlockSpec((1, OUT), lambda: (0, 0))],
        out_specs=[pl.BlockSpec((NUM_GRAPHS, OUT), lambda: (0, 0)),
                   pl.BlockSpec((NUM_GRAPHS, D), lambda: (0, 0))],
        out_shape=[jax.ShapeDtypeStruct((NUM_GRAPHS, OUT), jnp.float32),
                   jax.ShapeDtypeStruct((NUM_GRAPHS, D), jnp.float32)],
    )(h, batch2d, w_out, b_out.reshape(1, OUT))


def kernel(x, edge_index, batch, W1_0, b1_0, W2_0, b2_0, W1_1, b1_1, W2_1,
           b2_1, W1_2, b1_2, W2_2, b2_2, W_out, b_out):
    src = edge_index[0].reshape(NW, ROWS_PER_W, CHUNK)
    dst = edge_index[1].reshape(NW, ROWS_PER_W, CHUNK)
    batch2d = batch.reshape(N, 1)
    layer_params = [(W1_0, b1_0, W2_0, b2_0), (W1_1, b1_1, W2_1, b2_1),
                    (W1_2, b1_2, W2_2, b2_2)]
    h = x
    for l, (W1, b1, W2, b2) in enumerate(layer_params):
        parts = _seg_sum(h, src, dst)
        h = _mlp(h, parts[0, :N], parts[1, :N], W1, b1, W2, b2,
                 relu_out=(l < 2))
    out, emb = _pool(h, batch2d, W_out, b_out)
    return (out, emb)


# speedup vs baseline: 8.5339x; 1.3250x over previous
import functools

import jax
import jax.numpy as jnp
from jax import lax
from jax.experimental import pallas as pl
from jax.experimental.pallas import tpu as pltpu
from jax.experimental.pallas import tpu_sc as plsc

N = 10000
E = 320000
D = 128
OUT = 128
NUM_GRAPHS = 64

NC = 2
NS = 16
NW = NC * NS
CHUNK = 100
ROWS_PER_W = E // CHUNK // NW
NBLK = 5
RPB = ROWS_PER_W // NBLK
N_PAD = 10240
NODES_PER_S = N_PAD // NS


def _seg_sum_body(h_hbm, src_hbm, dst_hbm, out_hbm,
                  src_v, dst_v, rows_a, rows_b, agg_s, sem_a, sem_b):
    c = lax.axis_index("c")
    s = lax.axis_index("s")
    wid = c * NS + s

    zvec = jnp.zeros((16,), jnp.float32)
    def _zrow(i, carry):
        for j in range(D // 16):
            rows_a[i, pl.ds(j * 16, 16)] = zvec
        return carry
    lax.fori_loop(0, CHUNK, _zrow, 0)
    base = s * NODES_PER_S
    ZROWS = 80
    for k in range(NODES_PER_S // ZROWS):
        pltpu.sync_copy(rows_a.at[pl.ds(0, ZROWS)],
                        agg_s.at[pl.ds(base + k * ZROWS, ZROWS)])
    plsc.subcore_barrier()

    for b in range(NBLK):
        pltpu.sync_copy(src_hbm.at[wid, b], src_v)
        pltpu.sync_copy(dst_hbm.at[wid, b], dst_v)
        pltpu.async_copy(h_hbm.at[src_v.at[0]], rows_a, sem_a)
        def _pair(g, carry):
            j = 2 * g
            pltpu.make_async_copy(h_hbm.at[src_v.at[j]], rows_a, sem_a).wait()
            pltpu.async_copy(h_hbm.at[src_v.at[j + 1]], rows_b, sem_b)
            pltpu.sync_copy(rows_a, agg_s.at[dst_v.at[j]], add=True)
            pltpu.make_async_copy(h_hbm.at[src_v.at[j + 1]], rows_b, sem_b).wait()
            @pl.when(j + 2 < RPB)
            def _():
                pltpu.async_copy(h_hbm.at[src_v.at[j + 2]], rows_a, sem_a)
            pltpu.sync_copy(rows_b, agg_s.at[dst_v.at[j + 1]], add=True)
            return carry
        lax.fori_loop(0, RPB // 2, _pair, 0)
    plsc.subcore_barrier()

    pltpu.sync_copy(agg_s.at[pl.ds(s * NODES_PER_S, NODES_PER_S)],
                    out_hbm.at[c, pl.ds(s * NODES_PER_S, NODES_PER_S)])


_seg_sum = pl.kernel(
    _seg_sum_body,
    out_type=jax.ShapeDtypeStruct((NC, N_PAD, D), jnp.float32),
    mesh=plsc.VectorSubcoreMesh(core_axis_name="c", subcore_axis_name="s"),
    scratch_types=[
        pltpu.VMEM((RPB, CHUNK), jnp.int32),
        pltpu.VMEM((RPB, CHUNK), jnp.int32),
        pltpu.VMEM((CHUNK, D), jnp.float32),
        pltpu.VMEM((CHUNK, D), jnp.float32),
        pltpu.VMEM_SHARED((N_PAD, D), jnp.float32),
        pltpu.SemaphoreType.DMA,
        pltpu.SemaphoreType.DMA,
    ],
)


def _mlp_body(h_ref, p0_ref, p1_ref, w1_ref, b1_ref, w2_ref, b2_ref, o_ref,
              *, relu_out):
    m = h_ref[...] + p0_ref[...] + p1_ref[...]
    z = jnp.maximum(
        jnp.dot(m, w1_ref[...], preferred_element_type=jnp.float32)
        + b1_ref[...], 0.0)
    y = (jnp.dot(z, w2_ref[...], preferred_element_type=jnp.float32)
         + b2_ref[...])
    o_ref[...] = jnp.maximum(y, 0.0) if relu_out else y


BLK = 1000


def _mlp(h, p0, p1, w1, b1, w2, b2, relu_out):
    grid = (N // BLK,)
    row_spec = pl.BlockSpec((BLK, D), lambda i: (i, 0))
    full = lambda shape: pl.BlockSpec(shape, lambda i: (0, 0))
    return pl.pallas_call(
        functools.partial(_mlp_body, relu_out=relu_out),
        grid=grid,
        in_specs=[row_spec, row_spec, row_spec,
                  full((D, D)), full((1, D)), full((D, D)), full((1, D))],
        out_specs=row_spec,
        out_shape=jax.ShapeDtypeStruct((N, D), jnp.float32),
    )(h, p0, p1, w1, b1.reshape(1, D), w2, b2.reshape(1, D))


def _pool_body(h_ref, batch_ref, wout_ref, bout_ref, out_ref, emb_ref):
    oh = (batch_ref[...]
          == lax.broadcasted_iota(jnp.int32, (1, NUM_GRAPHS), 1)
          ).astype(jnp.float32)
    emb = lax.dot_general(oh, h_ref[...], (((0,), (0,)), ((), ())),
                          preferred_element_type=jnp.float32)
    emb_ref[...] = emb
    out_ref[...] = (jnp.dot(emb, wout_ref[...],
                            preferred_element_type=jnp.float32)
                    + bout_ref[...])


def _pool(h, batch2d, w_out, b_out):
    return pl.pallas_call(
        _pool_body,
        in_specs=[pl.BlockSpec((N, D), lambda: (0, 0)),
                  pl.BlockSpec((N, 1), lambda: (0, 0)),
                  pl.BlockSpec((D, OUT), lambda: (0, 0)),
                  pl.BlockSpec((1, OUT), lambda: (0, 0))],
        out_specs=[pl.BlockSpec((NUM_GRAPHS, OUT), lambda: (0, 0)),
                   pl.BlockSpec((NUM_GRAPHS, D), lambda: (0, 0))],
        out_shape=[jax.ShapeDtypeStruct((NUM_GRAPHS, OUT), jnp.float32),
                   jax.ShapeDtypeStruct((NUM_GRAPHS, D), jnp.float32)],
    )(h, batch2d, w_out, b_out.reshape(1, OUT))


def kernel(x, edge_index, batch, W1_0, b1_0, W2_0, b2_0, W1_1, b1_1, W2_1,
           b2_1, W1_2, b1_2, W2_2, b2_2, W_out, b_out):
    src = edge_index[0].reshape(NW, NBLK, RPB, CHUNK)
    dst = edge_index[1].reshape(NW, NBLK, RPB, CHUNK)
    batch2d = batch.reshape(N, 1)
    layer_params = [(W1_0, b1_0, W2_0, b2_0), (W1_1, b1_1, W2_1, b2_1),
                    (W1_2, b1_2, W2_2, b2_2)]
    h = x
    for l, (W1, b1, W2, b2) in enumerate(layer_params):
        parts = _seg_sum(h, src, dst)
        h = _mlp(h, parts[0, :N], parts[1, :N], W1, b1, W2, b2,
                 relu_out=(l < 2))
    out, emb = _pool(h, batch2d, W_out, b_out)
    return (out, emb)
